# Initial kernel scaffold; baseline (speedup 1.0000x reference)
import jax
import jax.numpy as jnp
from jax.experimental import pallas as pl


def kernel(gobyGenotypeIndex, isIndel, matchesReference, fromSequence, toSequence, genotypeCountForwardStrand, genotypeCountReverseStrand, geno_table, count_table, base_table, Wx, Wh, b_lstm, W_red, b_red):
    raise NotImplementedError("write your pallas kernel here")


# speedup vs baseline: 4.9888x; 4.9888x over previous
import functools

import jax
import jax.numpy as jnp
from jax import lax
from jax.experimental import pallas as pl
from jax.experimental.pallas import tpu as pltpu
from jax.experimental.pallas import tpu_sc as plsc

B = 4096
L = 50
H = 64
NIDX = 2 * B
CW = 8
NW = 32
BPW = NIDX // NW
BBLK = 512
N2 = 2 * BBLK
KOH = 88
KG = 104

@functools.cache
def _make_sc_count_gather():
    mesh = plsc.VectorSubcoreMesh(core_axis_name="c", subcore_axis_name="s")

    @functools.partial(
        pl.kernel,
        mesh=mesh,
        compiler_params=pltpu.CompilerParams(use_tc_tiling_on_sc=False),
        out_type=jax.ShapeDtypeStruct((NIDX, CW), jnp.float32),
        scratch_types=[
            pltpu.VMEM((BPW,), jnp.int32),
            pltpu.VMEM((BPW, CW), jnp.float32),
            pltpu.SemaphoreType.DMA,
        ],
    )
    def sc_count_gather(table_hbm, idx_hbm, out_hbm, idx_v, rows_v, sem):
        wid = lax.axis_index("s") * 2 + lax.axis_index("c")
        base = wid * BPW
        pltpu.sync_copy(idx_hbm.at[pl.ds(base, BPW)], idx_v)
        pltpu.async_copy(table_hbm.at[idx_v], rows_v, sem).wait()
        pltpu.sync_copy(rows_v, out_hbm.at[pl.ds(base, BPW)])

    return sc_count_gather


def _tc_body(seqf_ref, seqt_ref, geno_ref, ind_ref, mat_ref, cf_ref, cr_ref,
             WxT_ref, basepT_ref, WhT_ref, blT_ref, genopT_ref,
             WgT_ref, WiT_ref, WmT_ref, WfT_ref, WtT_ref,
             WcfT_ref, WcrT_ref, brT_ref, out_ref):
    f32 = jnp.float32
    XTt = jnp.dot(WxT_ref[...], basepT_ref[...],
                  preferred_element_type=f32) + blT_ref[...]
    Mg = jnp.dot(WgT_ref[...], genopT_ref[...],
                 preferred_element_type=f32)
    WhTv = WhT_ref[...]

    def step(t, hc):
        h, c = hc
        idx = jnp.concatenate(
            [seqf_ref[pl.ds(t, 1), :], seqt_ref[pl.ds(t, 1), :]], axis=1)
        oh = (lax.broadcasted_iota(jnp.int32, (KOH, N2), 0)
              == idx).astype(f32)
        z = (jnp.dot(XTt, oh, preferred_element_type=f32)
             + jnp.dot(WhTv, h, preferred_element_type=f32))
        i = jax.nn.sigmoid(z[0:H])
        f = jax.nn.sigmoid(z[H:2 * H])
        g = jnp.tanh(z[2 * H:3 * H])
        o = jax.nn.sigmoid(z[3 * H:4 * H])
        c = f * c + i * g
        h = o * jnp.tanh(c)
        return (h, c)

    h0 = jnp.zeros((H, N2), f32)
    c0 = jnp.zeros((H, N2), f32)
    h, _ = lax.fori_loop(0, L, step, (h0, c0))
    hf = h[:, :BBLK]
    ht = h[:, BBLK:]
    ohg = (lax.broadcasted_iota(jnp.int32, (KG, BBLK), 0)
           == geno_ref[...]).astype(f32)
    acc = (jnp.dot(Mg, ohg, preferred_element_type=f32)
           + jnp.dot(WfT_ref[...], hf, preferred_element_type=f32)
           + jnp.dot(WtT_ref[...], ht, preferred_element_type=f32)
           + jnp.dot(WcfT_ref[...], cf_ref[...], preferred_element_type=f32)
           + jnp.dot(WcrT_ref[...], cr_ref[...], preferred_element_type=f32)
           + brT_ref[...]
           + jnp.where(ind_ref[...] == 0, WiT_ref[:, 0:1], WiT_ref[:, 1:2])
           + jnp.where(mat_ref[...] == 0, WmT_ref[:, 0:1], WmT_ref[:, 1:2]))
    out_ref[...] = jnp.maximum(acc, 0.0)


def _tc_forward(seq_fT, seq_tT, genoT, indT, matT, cfT, crT,
                WxT, base_pT, WhT, blT, geno_pT,
                WgT, WiT, WmT, WfT, WtT, WcfT, WcrT, brT):
    nblk = B // BBLK
    bspec = lambda shp, im: pl.BlockSpec(shp, im)
    col = lambda j: (0, j)
    full = lambda j: (0, 0)
    in_specs = [
        bspec((L, BBLK), col), bspec((L, BBLK), col),
        bspec((1, BBLK), col), bspec((1, BBLK), col), bspec((1, BBLK), col),
        bspec((CW, BBLK), col), bspec((CW, BBLK), col),
        bspec((4 * H, 6), full), bspec((6, KOH), full),
        bspec((4 * H, H), full), bspec((4 * H, 1), full),
        bspec((4, KG), full),
        bspec((H, 4), full), bspec((H, 2), full), bspec((H, 2), full),
        bspec((H, H), full), bspec((H, H), full),
        bspec((H, CW), full), bspec((H, CW), full), bspec((H, 1), full),
    ]
    return pl.pallas_call(
        _tc_body,
        grid=(nblk,),
        in_specs=in_specs,
        out_specs=pl.BlockSpec((H, BBLK), col),
        out_shape=jax.ShapeDtypeStruct((H, B), jnp.float32),
    )(seq_fT, seq_tT, genoT, indT, matT, cfT, crT,
      WxT, base_pT, WhT, blT, geno_pT,
      WgT, WiT, WmT, WfT, WtT, WcfT, WcrT, brT)


def kernel(gobyGenotypeIndex, isIndel, matchesReference, fromSequence,
           toSequence, genotypeCountForwardStrand, genotypeCountReverseStrand,
           geno_table, count_table, base_table, Wx, Wh, b_lstm, W_red, b_red):
    i32 = jnp.int32
    f32 = jnp.float32
    idx_counts = jnp.concatenate(
        [genotypeCountForwardStrand, genotypeCountReverseStrand]).astype(i32)
    tbl8 = jnp.pad(count_table.astype(f32), ((0, 0), (0, CW - 5)))
    counts8 = _make_sc_count_gather()(tbl8, idx_counts)
    countsT = counts8.T
    cfT, crT = countsT[:, :B], countsT[:, B:]

    seq_fT = fromSequence.astype(i32).T
    seq_tT = toSequence.astype(i32).T
    genoT = gobyGenotypeIndex.astype(i32).reshape(1, B)
    indT = isIndel.astype(i32).reshape(1, B)
    matT = matchesReference.astype(i32).reshape(1, B)
    WxT = Wx.astype(f32).T
    base_pT = jnp.pad(base_table.astype(f32).T, ((0, 0), (0, KOH - 85)))
    WhT = Wh.astype(f32).T
    blT = b_lstm.astype(f32).reshape(4 * H, 1)
    geno_pT = jnp.pad(geno_table.astype(f32).T, ((0, 0), (0, KG - 100)))
    Wr = W_red.astype(f32)
    WgT = Wr[0:4].T
    WiT = Wr[4:6].T
    WmT = Wr[6:8].T
    WfT = Wr[8:72].T
    WtT = Wr[72:136].T
    WcfT = jnp.pad(Wr[136:141], ((0, CW - 5), (0, 0))).T
    WcrT = jnp.pad(Wr[141:146], ((0, CW - 5), (0, 0))).T
    brT = b_red.astype(f32).reshape(H, 1)

    outT = _tc_forward(seq_fT, seq_tT, genoT, indT, matT, cfT, crT,
                       WxT, base_pT, WhT, blT, geno_pT,
                       WgT, WiT, WmT, WfT, WtT, WcfT, WcrT, brT)
    return outT.T


# speedup vs baseline: 5.1040x; 1.0231x over previous
import functools

import jax
import jax.numpy as jnp
from jax import lax
from jax.experimental import pallas as pl
from jax.experimental.pallas import tpu as pltpu
from jax.experimental.pallas import tpu_sc as plsc

B = 4096
L = 50
H = 64
NIDX = 2 * B
CW = 8
NW = 32
BPW = NIDX // NW
BBLK = 512
N2 = 2 * BBLK
KOH = 88
KG = 104

@functools.cache
def _make_sc_count_gather():
    mesh = plsc.VectorSubcoreMesh(core_axis_name="c", subcore_axis_name="s")

    @functools.partial(
        pl.kernel,
        mesh=mesh,
        compiler_params=pltpu.CompilerParams(use_tc_tiling_on_sc=False),
        out_type=jax.ShapeDtypeStruct((NIDX, CW), jnp.float32),
        scratch_types=[
            pltpu.VMEM((BPW,), jnp.int32),
            pltpu.VMEM((BPW, CW), jnp.float32),
            pltpu.SemaphoreType.DMA,
        ],
    )
    def sc_count_gather(table_hbm, idx_hbm, out_hbm, idx_v, rows_v, sem):
        wid = lax.axis_index("s") * 2 + lax.axis_index("c")
        base = wid * BPW
        pltpu.sync_copy(idx_hbm.at[pl.ds(base, BPW)], idx_v)
        pltpu.async_copy(table_hbm.at[idx_v], rows_v, sem).wait()
        pltpu.sync_copy(rows_v, out_hbm.at[pl.ds(base, BPW)])

    return sc_count_gather


CH = 10


def _sig(x):
    return 0.5 * jnp.tanh(0.5 * x) + 0.5


def _tc_body(seqf_ref, seqt_ref, geno_ref, ind_ref, mat_ref, cf_ref, cr_ref,
             WxT_ref, basepT_ref, WhT_ref, blT_ref, genopT_ref,
             WgT_ref, WiT_ref, WmT_ref, WfT_ref, WtT_ref,
             WcfT_ref, WcrT_ref, brT_ref, out_ref, zx_ref):
    f32 = jnp.float32
    XTt = jnp.dot(WxT_ref[...], basepT_ref[...],
                  preferred_element_type=f32) + blT_ref[...]
    Mg = jnp.dot(WgT_ref[...], genopT_ref[...],
                 preferred_element_type=f32)
    WhTv = WhT_ref[...]
    iota = lax.broadcasted_iota(jnp.int32, (KOH, N2), 0)

    def outer(cc, hc):
        t0 = cc * CH
        for k in range(CH):
            idx = jnp.concatenate(
                [seqf_ref[pl.ds(t0 + k, 1), :],
                 seqt_ref[pl.ds(t0 + k, 1), :]], axis=1)
            oh = (iota == idx).astype(f32)
            zx_ref[k] = jnp.dot(XTt, oh, preferred_element_type=f32)

        def step(k, hc):
            h, c = hc
            z = zx_ref[k] + jnp.dot(WhTv, h, preferred_element_type=f32)
            i = _sig(z[0:H])
            f = _sig(z[H:2 * H])
            g = jnp.tanh(z[2 * H:3 * H])
            o = _sig(z[3 * H:4 * H])
            c = f * c + i * g
            h = o * jnp.tanh(c)
            return (h, c)

        return lax.fori_loop(0, CH, step, hc)

    h0 = jnp.zeros((H, N2), f32)
    c0 = jnp.zeros((H, N2), f32)
    h, _ = lax.fori_loop(0, L // CH, outer, (h0, c0))
    hf = h[:, :BBLK]
    ht = h[:, BBLK:]
    ohg = (lax.broadcasted_iota(jnp.int32, (KG, BBLK), 0)
           == geno_ref[...]).astype(f32)
    acc = (jnp.dot(Mg, ohg, preferred_element_type=f32)
           + jnp.dot(WfT_ref[...], hf, preferred_element_type=f32)
           + jnp.dot(WtT_ref[...], ht, preferred_element_type=f32)
           + jnp.dot(WcfT_ref[...], cf_ref[...], preferred_element_type=f32)
           + jnp.dot(WcrT_ref[...], cr_ref[...], preferred_element_type=f32)
           + brT_ref[...]
           + jnp.where(ind_ref[...] == 0, WiT_ref[:, 0:1], WiT_ref[:, 1:2])
           + jnp.where(mat_ref[...] == 0, WmT_ref[:, 0:1], WmT_ref[:, 1:2]))
    out_ref[...] = jnp.maximum(acc, 0.0)


def _tc_forward(seq_fT, seq_tT, genoT, indT, matT, cfT, crT,
                WxT, base_pT, WhT, blT, geno_pT,
                WgT, WiT, WmT, WfT, WtT, WcfT, WcrT, brT):
    nblk = B // BBLK
    bspec = lambda shp, im: pl.BlockSpec(shp, im)
    col = lambda j: (0, j)
    full = lambda j: (0, 0)
    in_specs = [
        bspec((L, BBLK), col), bspec((L, BBLK), col),
        bspec((1, BBLK), col), bspec((1, BBLK), col), bspec((1, BBLK), col),
        bspec((CW, BBLK), col), bspec((CW, BBLK), col),
        bspec((4 * H, 6), full), bspec((6, KOH), full),
        bspec((4 * H, H), full), bspec((4 * H, 1), full),
        bspec((4, KG), full),
        bspec((H, 4), full), bspec((H, 2), full), bspec((H, 2), full),
        bspec((H, H), full), bspec((H, H), full),
        bspec((H, CW), full), bspec((H, CW), full), bspec((H, 1), full),
    ]
    return pl.pallas_call(
        _tc_body,
        grid=(nblk,),
        in_specs=in_specs,
        out_specs=pl.BlockSpec((H, BBLK), col),
        out_shape=jax.ShapeDtypeStruct((H, B), jnp.float32),
        scratch_shapes=[pltpu.VMEM((CH, 4 * H, N2), jnp.float32)],
    )(seq_fT, seq_tT, genoT, indT, matT, cfT, crT,
      WxT, base_pT, WhT, blT, geno_pT,
      WgT, WiT, WmT, WfT, WtT, WcfT, WcrT, brT)


def kernel(gobyGenotypeIndex, isIndel, matchesReference, fromSequence,
           toSequence, genotypeCountForwardStrand, genotypeCountReverseStrand,
           geno_table, count_table, base_table, Wx, Wh, b_lstm, W_red, b_red):
    i32 = jnp.int32
    f32 = jnp.float32
    idx_counts = jnp.concatenate(
        [genotypeCountForwardStrand, genotypeCountReverseStrand]).astype(i32)
    tbl8 = jnp.pad(count_table.astype(f32), ((0, 0), (0, CW - 5)))
    counts8 = _make_sc_count_gather()(tbl8, idx_counts)
    countsT = counts8.T
    cfT, crT = countsT[:, :B], countsT[:, B:]

    seq_fT = fromSequence.astype(i32).T
    seq_tT = toSequence.astype(i32).T
    genoT = gobyGenotypeIndex.astype(i32).reshape(1, B)
    indT = isIndel.astype(i32).reshape(1, B)
    matT = matchesReference.astype(i32).reshape(1, B)
    WxT = Wx.astype(f32).T
    base_pT = jnp.pad(base_table.astype(f32).T, ((0, 0), (0, KOH - 85)))
    WhT = Wh.astype(f32).T
    blT = b_lstm.astype(f32).reshape(4 * H, 1)
    geno_pT = jnp.pad(geno_table.astype(f32).T, ((0, 0), (0, KG - 100)))
    Wr = W_red.astype(f32)
    WgT = Wr[0:4].T
    WiT = Wr[4:6].T
    WmT = Wr[6:8].T
    WfT = Wr[8:72].T
    WtT = Wr[72:136].T
    WcfT = jnp.pad(Wr[136:141], ((0, CW - 5), (0, 0))).T
    WcrT = jnp.pad(Wr[141:146], ((0, CW - 5), (0, 0))).T
    brT = b_red.astype(f32).reshape(H, 1)

    outT = _tc_forward(seq_fT, seq_tT, genoT, indT, matT, cfT, crT,
                       WxT, base_pT, WhT, blT, geno_pT,
                       WgT, WiT, WmT, WfT, WtT, WcfT, WcrT, brT)
    return outT.T
